# baseline (speedup 1.0000x reference)
import functools

import jax
import jax.numpy as jnp
from jax import lax
from jax.experimental import pallas as pl
from jax.experimental.pallas import tpu as pltpu
from jax.experimental.pallas import tpu_sc as plsc

BATCH = 16384
EMBED_DIM = 64
NUM_VARIETIES = 1000000
LANES = 16
NC = 2
NS = 16
NW = NC * NS
B_PER_W = BATCH // NW
N_BLOCKS = NUM_VARIETIES // 128
TAIL_BLK = N_BLOCKS
TAIL_START = N_BLOCKS * 128
Q = EMBED_DIM // LANES


NO_SWITCH = -1
NO_PREFETCH = 0x7FFFFFF


def _sc_variety_gather(vt_T, tail_blk, sidx, enc, dest):
    mesh = plsc.VectorSubcoreMesh(core_axis_name="c", subcore_axis_name="s")

    @functools.partial(
        pl.kernel,
        mesh=mesh,
        compiler_params=pltpu.CompilerParams(needs_layout_passes=False),
        out_type=[pltpu.HBM((BATCH * EMBED_DIM,), jnp.float32)],
        scratch_types=[
            pltpu.VMEM((B_PER_W,), jnp.int32),
            pltpu.VMEM((B_PER_W,), jnp.int32),
            pltpu.VMEM((B_PER_W,), jnp.int32),
            pltpu.VMEM((2, EMBED_DIM, 128), jnp.float32),
            pltpu.VMEM((B_PER_W * EMBED_DIM,), jnp.float32),
            pltpu.SemaphoreType.DMA,
            pltpu.SemaphoreType.DMA,
        ],
    )
    def var_kernel(vt_hbm, tail_hbm, sidx_hbm, enc_hbm, dest_hbm, out_hbm,
                   sidx_v, enc_v, dest_v, blk2_v, rows_v, bsem, osem):
        wid = lax.axis_index("s") * NC + lax.axis_index("c")
        base = wid * B_PER_W
        pltpu.sync_copy(sidx_hbm.at[pl.ds(base, B_PER_W)], sidx_v)
        pltpu.sync_copy(enc_hbm.at[pl.ds(base, B_PER_W)], enc_v)
        pltpu.sync_copy(dest_hbm.at[pl.ds(base, B_PER_W)], dest_v)

        def start_block_dma(b, slot):
            @pl.when(b < TAIL_BLK)
            def _():
                pltpu.async_copy(vt_hbm.at[:, pl.ds(b * 128, 128)],
                                 blk2_v.at[slot], bsem)

            @pl.when(b >= TAIL_BLK)
            def _():
                pltpu.async_copy(tail_hbm, blk2_v.at[slot], bsem)

        first_spl = plsc.load_gather(sidx_v, [jnp.zeros((LANES,), jnp.int32)])
        start_block_dma(first_spl[0] >> 7, 0)

        def item_body(j, carry):
            slot = carry
            jf = jnp.full((LANES,), j, jnp.int32)
            sidx_spl = plsc.load_gather(sidx_v, [jf])
            e = plsc.load_gather(enc_v, [jf])[0]
            is_switch = e != NO_SWITCH
            slot2 = jnp.where(is_switch, 1 - slot, slot)

            @pl.when(is_switch)
            def _():
                pltpu.make_async_copy(
                    tail_hbm, blk2_v.at[slot2], bsem).wait()

                @pl.when(e != NO_PREFETCH)
                def _():
                    start_block_dma(e, 1 - slot2)

            lane_spl = sidx_spl & 127
            slot_spl = jnp.full((LANES,), slot2, jnp.int32)
            for q in range(Q):
                ridx = lax.broadcasted_iota(jnp.int32, (LANES,), 0) + q * LANES
                vals = plsc.load_gather(blk2_v, [slot_spl, ridx, lane_spl])
                rows_v[pl.ds(j * EMBED_DIM + q * LANES, LANES)] = vals
            dest_s = plsc.load_gather(dest_v, [jf])[0]
            pltpu.async_copy(
                rows_v.at[pl.ds(j * EMBED_DIM, EMBED_DIM)],
                out_hbm.at[pl.ds(dest_s * EMBED_DIM, EMBED_DIM)], osem)
            return slot2

        pl.loop(0, B_PER_W, init_carry=jnp.int32(1))(item_body)

        @pl.loop(0, B_PER_W)
        def drain(j):
            pltpu.make_async_copy(
                out_hbm.at[pl.ds(0, EMBED_DIM)],
                rows_v.at[pl.ds(0, EMBED_DIM)], osem).wait()

    return var_kernel(vt_T, tail_blk, sidx, enc, dest)[0]


def _sc_technique_gather(technique_table, tidx2d):
    mesh = plsc.VectorSubcoreMesh(core_axis_name="c", subcore_axis_name="s")
    CHUNK = 128
    N_CHUNKS = B_PER_W // CHUNK

    @functools.partial(
        pl.kernel,
        mesh=mesh,
        compiler_params=pltpu.CompilerParams(use_tc_tiling_on_sc=False),
        out_type=[jax.ShapeDtypeStruct((BATCH, EMBED_DIM), jnp.float32)],
        scratch_types=[
            pltpu.VMEM((N_CHUNKS, CHUNK), jnp.int32),
            pltpu.VMEM((B_PER_W, EMBED_DIM), jnp.float32),
            pltpu.SemaphoreType.DMA,
        ],
    )
    def tech_kernel(tt_hbm, tidx_hbm, tout_hbm, tidx_v, trows_v, tsem):
        wid = lax.axis_index("s") * NC + lax.axis_index("c")
        row0 = wid * N_CHUNKS
        pltpu.sync_copy(tidx_hbm.at[pl.ds(row0, N_CHUNKS)], tidx_v)
        copies = []
        for j in range(N_CHUNKS):
            copies.append(pltpu.async_copy(
                tt_hbm.at[tidx_v.at[j]],
                trows_v.at[pl.ds(j * CHUNK, CHUNK)], tsem))
        for c in copies:
            c.wait()
        base = wid * B_PER_W
        pltpu.sync_copy(trows_v, tout_hbm.at[pl.ds(base, B_PER_W)])

    return tech_kernel(technique_table, tidx2d)[0]


def _gelu_exact(x):
    return 0.5 * x * (1.0 + lax.erf(x * 0.7071067811865476))


def _combine_body(ve_ref, te_ref, cont_ref, wc_ref, bc_ref, wcomb_ref,
                  bcomb_ref, out_ref):
    p = jnp.dot(cont_ref[...], wc_ref[...],
                preferred_element_type=jnp.float32) + bc_ref[...]
    p = _gelu_exact(p)
    comb = jnp.concatenate([ve_ref[...], te_ref[...], p], axis=-1)
    z = jnp.dot(comb, wcomb_ref[...],
                preferred_element_type=jnp.float32) + bcomb_ref[...]
    out_ref[...] = _gelu_exact(z)


def _tc_combine(ve, te, cont, W_cont, b_cont, W_comb, b_comb):
    blk = 1024
    grid = (BATCH // blk,)
    bspec = pl.BlockSpec((blk, EMBED_DIM), lambda i: (i, 0))
    full = lambda shape: pl.BlockSpec(shape, lambda i: (0, 0))
    return pl.pallas_call(
        _combine_body,
        grid=grid,
        in_specs=[
            bspec, bspec, bspec,
            full((EMBED_DIM, EMBED_DIM)),
            full((1, EMBED_DIM)),
            full((3 * EMBED_DIM, EMBED_DIM)),
            full((1, EMBED_DIM)),
        ],
        out_specs=bspec,
        out_shape=jax.ShapeDtypeStruct((BATCH, EMBED_DIM), jnp.float32),
    )(ve, te, cont, W_cont, b_cont, W_comb, b_comb)


def kernel(variety, technique, continuous, variety_table, technique_table,
           W_cont, b_cont, W_comb, b_comb):
    order = jnp.argsort(variety).astype(jnp.int32)
    sidx = jnp.take(variety, order)
    sblk = sidx >> 7
    pos = jnp.arange(BATCH, dtype=jnp.int32)
    prev = jnp.concatenate([sblk[:1] - 1, sblk[:-1]])
    is_first = (sblk != prev) | (pos % B_PER_W == 0)
    nxt_pos = jnp.searchsorted(sblk, sblk, side="right").astype(jnp.int32)
    worker_end = (pos // B_PER_W) * B_PER_W + (B_PER_W - 1)
    nd_val = sblk[jnp.clip(nxt_pos, 0, BATCH - 1)]
    enc = jnp.where(~is_first, NO_SWITCH,
                    jnp.where(nxt_pos <= worker_end, nd_val, NO_PREFETCH))
    vt_T = variety_table.T
    tail = jnp.pad(variety_table[TAIL_START:], ((0, 64), (0, 0))).T
    ve_flat = _sc_variety_gather(vt_T, tail, sidx, enc, order)
    ve = ve_flat.reshape(BATCH, EMBED_DIM)
    te = _sc_technique_gather(technique_table,
                              technique.reshape(BATCH // 128, 128))
    out = _tc_combine(ve, te, continuous,
                      W_cont, b_cont.reshape(1, EMBED_DIM),
                      W_comb, b_comb.reshape(1, EMBED_DIM))
    return out


# speedup vs baseline: 2.1049x; 2.1049x over previous
import functools

import jax
import jax.numpy as jnp
from jax import lax
from jax.experimental import pallas as pl
from jax.experimental.pallas import tpu as pltpu
from jax.experimental.pallas import tpu_sc as plsc

BATCH = 16384
EMBED_DIM = 64
NUM_VARIETIES = 1000000
LANES = 16
NC = 2
NS = 16
NW = NC * NS
B_PER_W = BATCH // NW
N_BLOCKS = NUM_VARIETIES // 128
TAIL_BLK = N_BLOCKS
TAIL_START = N_BLOCKS * 128
Q = EMBED_DIM // LANES


NO_SWITCH = -1
NO_PREFETCH = 0x7FFFFFF


def _sc_variety_gather(vt_T, tail_blk, sidx, enc, dest):
    mesh = plsc.VectorSubcoreMesh(core_axis_name="c", subcore_axis_name="s")

    @functools.partial(
        pl.kernel,
        mesh=mesh,
        compiler_params=pltpu.CompilerParams(needs_layout_passes=False),
        out_type=[pltpu.HBM((BATCH * EMBED_DIM,), jnp.float32)],
        scratch_types=[
            pltpu.VMEM((B_PER_W,), jnp.int32),
            pltpu.VMEM((B_PER_W,), jnp.int32),
            pltpu.VMEM((B_PER_W,), jnp.int32),
            pltpu.VMEM((2, EMBED_DIM, 128), jnp.float32),
            pltpu.VMEM((B_PER_W * EMBED_DIM,), jnp.float32),
            pltpu.SemaphoreType.DMA,
            pltpu.SemaphoreType.DMA,
        ],
    )
    def var_kernel(vt_hbm, tail_hbm, sidx_hbm, enc_hbm, dest_hbm, out_hbm,
                   sidx_v, enc_v, dest_v, blk2_v, rows_v, bsem, osem):
        wid = lax.axis_index("s") * NC + lax.axis_index("c")
        base = wid * B_PER_W
        pltpu.sync_copy(sidx_hbm.at[pl.ds(base, B_PER_W)], sidx_v)
        pltpu.sync_copy(enc_hbm.at[pl.ds(base, B_PER_W)], enc_v)
        pltpu.sync_copy(dest_hbm.at[pl.ds(base, B_PER_W)], dest_v)

        def start_block_dma(b, slot):
            @pl.when(b < TAIL_BLK)
            def _():
                pltpu.async_copy(vt_hbm.at[:, pl.ds(b * 128, 128)],
                                 blk2_v.at[slot], bsem)

            @pl.when(b >= TAIL_BLK)
            def _():
                pltpu.async_copy(tail_hbm, blk2_v.at[slot], bsem)

        first_spl = plsc.load_gather(sidx_v, [jnp.zeros((LANES,), jnp.int32)])
        start_block_dma(first_spl[0] >> 7, 0)

        def item_body(j, carry):
            slot = carry
            jf = jnp.full((LANES,), j, jnp.int32)
            sidx_spl = plsc.load_gather(sidx_v, [jf])
            e = plsc.load_gather(enc_v, [jf])[0]
            is_switch = e != NO_SWITCH
            slot2 = jnp.where(is_switch, 1 - slot, slot)

            @pl.when(is_switch)
            def _():
                pltpu.make_async_copy(
                    tail_hbm, blk2_v.at[slot2], bsem).wait()

                @pl.when(e != NO_PREFETCH)
                def _():
                    start_block_dma(e, 1 - slot2)

            lane_spl = sidx_spl & 127
            slot_spl = jnp.full((LANES,), slot2, jnp.int32)
            for q in range(Q):
                ridx = lax.broadcasted_iota(jnp.int32, (LANES,), 0) + q * LANES
                vals = plsc.load_gather(blk2_v, [slot_spl, ridx, lane_spl])
                rows_v[pl.ds(j * EMBED_DIM + q * LANES, LANES)] = vals
            dest_s = plsc.load_gather(dest_v, [jf])[0]
            pltpu.async_copy(
                rows_v.at[pl.ds(j * EMBED_DIM, EMBED_DIM)],
                out_hbm.at[pl.ds(dest_s * EMBED_DIM, EMBED_DIM)], osem)
            return slot2

        pl.loop(0, B_PER_W, init_carry=jnp.int32(1))(item_body)

        @pl.loop(0, B_PER_W)
        def drain(j):
            pltpu.make_async_copy(
                out_hbm.at[pl.ds(0, EMBED_DIM)],
                rows_v.at[pl.ds(0, EMBED_DIM)], osem).wait()

    return var_kernel(vt_T, tail_blk, sidx, enc, dest)[0]


def _sc_technique_gather(technique_table, tidx2d):
    mesh = plsc.VectorSubcoreMesh(core_axis_name="c", subcore_axis_name="s")
    CHUNK = 128
    N_CHUNKS = B_PER_W // CHUNK

    @functools.partial(
        pl.kernel,
        mesh=mesh,
        compiler_params=pltpu.CompilerParams(use_tc_tiling_on_sc=False),
        out_type=[jax.ShapeDtypeStruct((BATCH, EMBED_DIM), jnp.float32)],
        scratch_types=[
            pltpu.VMEM((N_CHUNKS, CHUNK), jnp.int32),
            pltpu.VMEM((B_PER_W, EMBED_DIM), jnp.float32),
            pltpu.SemaphoreType.DMA,
        ],
    )
    def tech_kernel(tt_hbm, tidx_hbm, tout_hbm, tidx_v, trows_v, tsem):
        wid = lax.axis_index("s") * NC + lax.axis_index("c")
        row0 = wid * N_CHUNKS
        pltpu.sync_copy(tidx_hbm.at[pl.ds(row0, N_CHUNKS)], tidx_v)
        copies = []
        for j in range(N_CHUNKS):
            copies.append(pltpu.async_copy(
                tt_hbm.at[tidx_v.at[j]],
                trows_v.at[pl.ds(j * CHUNK, CHUNK)], tsem))
        for c in copies:
            c.wait()
        base = wid * B_PER_W
        pltpu.sync_copy(trows_v, tout_hbm.at[pl.ds(base, B_PER_W)])

    return tech_kernel(technique_table, tidx2d)[0]


def _gelu_exact(x):
    return 0.5 * x * (1.0 + lax.erf(x * 0.7071067811865476))


def _combine_body(ve_ref, te_ref, cont_ref, wc_ref, bc_ref, wcomb_ref,
                  bcomb_ref, out_ref):
    p = jnp.dot(cont_ref[...], wc_ref[...],
                preferred_element_type=jnp.float32) + bc_ref[...]
    p = _gelu_exact(p)
    comb = jnp.concatenate([ve_ref[...], te_ref[...], p], axis=-1)
    z = jnp.dot(comb, wcomb_ref[...],
                preferred_element_type=jnp.float32) + bcomb_ref[...]
    out_ref[...] = _gelu_exact(z)


def _tc_combine(ve, te, cont, W_cont, b_cont, W_comb, b_comb):
    blk = 1024
    grid = (BATCH // blk,)
    bspec = pl.BlockSpec((blk, EMBED_DIM), lambda i: (i, 0))
    full = lambda shape: pl.BlockSpec(shape, lambda i: (0, 0))
    return pl.pallas_call(
        _combine_body,
        grid=grid,
        in_specs=[
            bspec, bspec, bspec,
            full((EMBED_DIM, EMBED_DIM)),
            full((1, EMBED_DIM)),
            full((3 * EMBED_DIM, EMBED_DIM)),
            full((1, EMBED_DIM)),
        ],
        out_specs=bspec,
        out_shape=jax.ShapeDtypeStruct((BATCH, EMBED_DIM), jnp.float32),
    )(ve, te, cont, W_cont, b_cont, W_comb, b_comb)


def kernel(variety, technique, continuous, variety_table, technique_table,
           W_cont, b_cont, W_comb, b_comb):
    pos = jnp.arange(BATCH, dtype=jnp.int32)
    order = jnp.argsort(variety).astype(jnp.int32)
    sidx = jnp.take(variety, order)
    sblk = sidx >> 7
    nxt_diff = sblk[1:] != sblk[:-1]
    BIG = jnp.int32(2 ** 30)
    posarr = jnp.concatenate(
        [jnp.where(nxt_diff, pos[1:], BIG), jnp.full((1,), BIG, jnp.int32)])
    blkarr = jnp.concatenate(
        [jnp.where(nxt_diff, sblk[1:], BIG), jnp.full((1,), BIG, jnp.int32)])
    nxt_pos = lax.cummin(posarr, axis=0, reverse=True)
    nd_blk = lax.cummin(blkarr, axis=0, reverse=True)
    is_first = jnp.concatenate(
        [jnp.ones((1,), jnp.bool_), nxt_diff]) | (pos % B_PER_W == 0)
    worker_end = (pos // B_PER_W) * B_PER_W + (B_PER_W - 1)
    enc = jnp.where(~is_first, NO_SWITCH,
                    jnp.where(nxt_pos <= worker_end, nd_blk, NO_PREFETCH))
    vt_T = variety_table.T
    tail = jnp.pad(variety_table[TAIL_START:], ((0, 64), (0, 0))).T
    ve_flat = _sc_variety_gather(vt_T, tail, sidx, enc, order)
    ve = ve_flat.reshape(BATCH, EMBED_DIM)
    te = _sc_technique_gather(technique_table,
                              technique.reshape(BATCH // 128, 128))
    out = _tc_combine(ve, te, continuous,
                      W_cont, b_cont.reshape(1, EMBED_DIM),
                      W_comb, b_comb.reshape(1, EMBED_DIM))
    return out


# speedup vs baseline: 2.1118x; 1.0033x over previous
import functools

import jax
import jax.numpy as jnp
from jax import lax
from jax.experimental import pallas as pl
from jax.experimental.pallas import tpu as pltpu
from jax.experimental.pallas import tpu_sc as plsc

BATCH = 16384
EMBED_DIM = 64
NUM_VARIETIES = 1000000
LANES = 16
NC = 2
NS = 16
NW = NC * NS
B_PER_W = BATCH // NW
N_BLOCKS = NUM_VARIETIES // 128
TAIL_BLK = N_BLOCKS
TAIL_START = N_BLOCKS * 128
Q = EMBED_DIM // LANES


NO_SWITCH = -1
NO_PREFETCH = 0x7FFFFFF


def _sc_variety_gather(vt_T, tail_blk, sidx, enc, dest):
    mesh = plsc.VectorSubcoreMesh(core_axis_name="c", subcore_axis_name="s")

    @functools.partial(
        pl.kernel,
        mesh=mesh,
        compiler_params=pltpu.CompilerParams(needs_layout_passes=False),
        out_type=[pltpu.HBM((BATCH * EMBED_DIM,), jnp.float32)],
        scratch_types=[
            pltpu.VMEM((B_PER_W,), jnp.int32),
            pltpu.VMEM((B_PER_W,), jnp.int32),
            pltpu.VMEM((B_PER_W,), jnp.int32),
            pltpu.VMEM((2, EMBED_DIM, 128), jnp.float32),
            pltpu.VMEM((B_PER_W * EMBED_DIM,), jnp.float32),
            pltpu.SemaphoreType.DMA,
            pltpu.SemaphoreType.DMA,
        ],
    )
    def var_kernel(vt_hbm, tail_hbm, sidx_hbm, enc_hbm, dest_hbm, out_hbm,
                   sidx_v, enc_v, dest_v, blk2_v, rows_v, bsem, osem):
        wid = lax.axis_index("s") * NC + lax.axis_index("c")
        base = wid * B_PER_W
        pltpu.sync_copy(sidx_hbm.at[pl.ds(base, B_PER_W)], sidx_v)
        pltpu.sync_copy(enc_hbm.at[pl.ds(base, B_PER_W)], enc_v)
        pltpu.sync_copy(dest_hbm.at[pl.ds(base, B_PER_W)], dest_v)

        def start_block_dma(b, slot):
            @pl.when(b < TAIL_BLK)
            def _():
                pltpu.async_copy(vt_hbm.at[:, pl.ds(b * 128, 128)],
                                 blk2_v.at[slot], bsem)

            @pl.when(b >= TAIL_BLK)
            def _():
                pltpu.async_copy(tail_hbm, blk2_v.at[slot], bsem)

        first_spl = plsc.load_gather(sidx_v, [jnp.zeros((LANES,), jnp.int32)])
        start_block_dma(first_spl[0] >> 7, 0)

        def item_body(j, carry):
            slot = carry
            jf = jnp.full((LANES,), j, jnp.int32)
            sidx_spl = plsc.load_gather(sidx_v, [jf])
            e = plsc.load_gather(enc_v, [jf])[0]
            is_switch = e != NO_SWITCH
            slot2 = jnp.where(is_switch, 1 - slot, slot)

            @pl.when(is_switch)
            def _():
                pltpu.make_async_copy(
                    tail_hbm, blk2_v.at[slot2], bsem).wait()

                @pl.when(e != NO_PREFETCH)
                def _():
                    start_block_dma(e, 1 - slot2)

            lane_spl = sidx_spl & 127
            slot_spl = jnp.full((LANES,), slot2, jnp.int32)
            for q in range(Q):
                ridx = lax.broadcasted_iota(jnp.int32, (LANES,), 0) + q * LANES
                vals = plsc.load_gather(blk2_v, [slot_spl, ridx, lane_spl])
                rows_v[pl.ds(j * EMBED_DIM + q * LANES, LANES)] = vals
            dest_s = plsc.load_gather(dest_v, [jf])[0]
            pltpu.async_copy(
                rows_v.at[pl.ds(j * EMBED_DIM, EMBED_DIM)],
                out_hbm.at[pl.ds(dest_s * EMBED_DIM, EMBED_DIM)], osem)
            return slot2

        pl.loop(0, B_PER_W, init_carry=jnp.int32(1))(item_body)

        @pl.loop(0, B_PER_W)
        def drain(j):
            pltpu.make_async_copy(
                out_hbm.at[pl.ds(0, EMBED_DIM)],
                rows_v.at[pl.ds(0, EMBED_DIM)], osem).wait()

    return var_kernel(vt_T, tail_blk, sidx, enc, dest)[0]


def _sc_technique_gather(technique_table, tidx2d):
    mesh = plsc.VectorSubcoreMesh(core_axis_name="c", subcore_axis_name="s")
    CHUNK = 128
    N_CHUNKS = B_PER_W // CHUNK

    @functools.partial(
        pl.kernel,
        mesh=mesh,
        compiler_params=pltpu.CompilerParams(use_tc_tiling_on_sc=False),
        out_type=[jax.ShapeDtypeStruct((BATCH, EMBED_DIM), jnp.float32)],
        scratch_types=[
            pltpu.VMEM((N_CHUNKS, CHUNK), jnp.int32),
            pltpu.VMEM((B_PER_W, EMBED_DIM), jnp.float32),
            pltpu.SemaphoreType.DMA,
        ],
    )
    def tech_kernel(tt_hbm, tidx_hbm, tout_hbm, tidx_v, trows_v, tsem):
        wid = lax.axis_index("s") * NC + lax.axis_index("c")
        row0 = wid * N_CHUNKS
        pltpu.sync_copy(tidx_hbm.at[pl.ds(row0, N_CHUNKS)], tidx_v)
        copies = []
        for j in range(N_CHUNKS):
            copies.append(pltpu.async_copy(
                tt_hbm.at[tidx_v.at[j]],
                trows_v.at[pl.ds(j * CHUNK, CHUNK)], tsem))
        for c in copies:
            c.wait()
        base = wid * B_PER_W
        pltpu.sync_copy(trows_v, tout_hbm.at[pl.ds(base, B_PER_W)])

    return tech_kernel(technique_table, tidx2d)[0]


def _gelu_exact(x):
    return 0.5 * x * (1.0 + lax.erf(x * 0.7071067811865476))


def _combine_body(ve_ref, te_ref, cont_ref, wc_ref, bc_ref, wcomb_ref,
                  bcomb_ref, out_ref):
    p = jnp.dot(cont_ref[...], wc_ref[...],
                preferred_element_type=jnp.float32) + bc_ref[...]
    p = _gelu_exact(p)
    comb = jnp.concatenate([ve_ref[...], te_ref[...], p], axis=-1)
    z = jnp.dot(comb, wcomb_ref[...],
                preferred_element_type=jnp.float32) + bcomb_ref[...]
    out_ref[...] = _gelu_exact(z)


def _tc_combine(ve, te, cont, W_cont, b_cont, W_comb, b_comb):
    blk = 1024
    grid = (BATCH // blk,)
    bspec = pl.BlockSpec((blk, EMBED_DIM), lambda i: (i, 0))
    full = lambda shape: pl.BlockSpec(shape, lambda i: (0, 0))
    return pl.pallas_call(
        _combine_body,
        grid=grid,
        in_specs=[
            bspec, bspec, bspec,
            full((EMBED_DIM, EMBED_DIM)),
            full((1, EMBED_DIM)),
            full((3 * EMBED_DIM, EMBED_DIM)),
            full((1, EMBED_DIM)),
        ],
        out_specs=bspec,
        out_shape=jax.ShapeDtypeStruct((BATCH, EMBED_DIM), jnp.float32),
    )(ve, te, cont, W_cont, b_cont, W_comb, b_comb)


def kernel(variety, technique, continuous, variety_table, technique_table,
           W_cont, b_cont, W_comb, b_comb):
    pos = jnp.arange(BATCH, dtype=jnp.int32)
    packed = ((variety >> 7) << 14) | pos
    spacked = jnp.sort(packed)
    order = spacked & (BATCH - 1)
    sidx = jnp.take(variety, order)
    sblk = spacked >> 14
    nxt_diff = sblk[1:] != sblk[:-1]
    BIG = jnp.int32(2 ** 30)
    posarr = jnp.concatenate(
        [jnp.where(nxt_diff, pos[1:], BIG), jnp.full((1,), BIG, jnp.int32)])
    blkarr = jnp.concatenate(
        [jnp.where(nxt_diff, sblk[1:], BIG), jnp.full((1,), BIG, jnp.int32)])
    nxt_pos = lax.cummin(posarr, axis=0, reverse=True)
    nd_blk = lax.cummin(blkarr, axis=0, reverse=True)
    is_first = jnp.concatenate(
        [jnp.ones((1,), jnp.bool_), nxt_diff]) | (pos % B_PER_W == 0)
    worker_end = (pos // B_PER_W) * B_PER_W + (B_PER_W - 1)
    enc = jnp.where(~is_first, NO_SWITCH,
                    jnp.where(nxt_pos <= worker_end, nd_blk, NO_PREFETCH))
    vt_T = variety_table.T
    tail = jnp.pad(variety_table[TAIL_START:], ((0, 64), (0, 0))).T
    ve_flat = _sc_variety_gather(vt_T, tail, sidx, enc, order)
    ve = ve_flat.reshape(BATCH, EMBED_DIM)
    te = _sc_technique_gather(technique_table,
                              technique.reshape(BATCH // 128, 128))
    out = _tc_combine(ve, te, continuous,
                      W_cont, b_cont.reshape(1, EMBED_DIM),
                      W_comb, b_comb.reshape(1, EMBED_DIM))
    return out
